# baseline (speedup 1.0000x reference)
import numpy as np
import jax
import jax.numpy as jnp
from jax.experimental import pallas as pl
from jax.experimental.pallas import tpu as pltpu

_N = 8192
_M = 2048
_B = 4
_C_IN = 128
_C_RED = 32
_C_LOCAL = 32
_GROUP_CFGS = (
    ((2, 2, 2), 0.2, (32, 32)),
    ((3, 3, 3), 0.4, (32, 32)),
)
_MULT = 2.0
_BN_EPS = 1e-5
_SB = _N // _B
_QB = _M // _B
_QC = 128
_INV = float(1.0 / np.sqrt(np.float32(1.0 + _BN_EPS)))
_BIG = np.float32(1e10)


def _grid_offsets_np(R, nv):
    gs = [np.arange(-R + R / n, R - R / n + 1e-5, 2 * R / n, dtype=np.float32)
          for n in nv]
    gx, gy, gz = np.meshgrid(gs[0], gs[1], gs[2], indexing='ij')
    return np.stack([gx.reshape(-1), gy.reshape(-1), gz.reshape(-1)], axis=-1)


def _body(xyzT_ref, xyz_ref, feat_ref, q_ref,
          Wsep0_ref, Wp00_ref, Wp01_ref, Wsep1_ref, Wp10_ref, Wp11_ref,
          Wmsg_ref, out_ref):
    f32 = jnp.float32
    sx = xyzT_ref[0:1, :]
    sy = xyzT_ref[1:2, :]
    sz = xyzT_ref[2:3, :]
    q = q_ref[...]
    qx = q[:, 0:1]
    qy = q[:, 1:2]
    qz = q[:, 2:3]
    feats = feat_ref[...]
    fred = (feats[:, 0:32] + feats[:, 32:64]
            + feats[:, 64:96] + feats[:, 96:128])
    gmat = jnp.concatenate([xyz_ref[...], fred], axis=1)

    cheb = jnp.maximum(jnp.maximum(jnp.abs(qx - sx), jnp.abs(qy - sy)),
                       jnp.abs(qz - sz))
    iota = jax.lax.broadcasted_iota(jnp.int32, (_QC, _SB), 1)

    group_outs = []
    wsep_refs = (Wsep0_ref, Wsep1_ref)
    wpost_refs = ((Wp00_ref, Wp01_ref), (Wp10_ref, Wp11_ref))
    for g, (nv, R, _post) in enumerate(_GROUP_CFGS):
        tv = nv[0] * nv[1] * nv[2]
        offs = _grid_offsets_np(R, nv)
        cand = cheb <= f32(R * _MULT)
        ovals = [float(o) for o in np.unique(offs[:, 0])]
        dxyz = (qx - sx, qy - sy, qz - sz)
        terms = []
        for d in dxyz:
            d2ax = d * d
            terms.append({o: d2ax + (f32(2.0 * o) * d + f32(o * o))
                          for o in ovals})
        outs_v = []
        for v in range(tv):
            ox = float(offs[v, 0])
            oy = float(offs[v, 1])
            oz = float(offs[v, 2])
            cx = qx + f32(ox)
            cy = qy + f32(oy)
            cz = qz + f32(oz)
            d2 = (terms[0][ox] + terms[1][oy]) + terms[2][oz]
            cur = jnp.where(cand, d2, _BIG)
            ohs = []
            ds = []
            for k in range(3):
                mk = jnp.min(cur, axis=1, keepdims=True)
                sel = jnp.min(jnp.where(cur == mk, iota, _SB),
                              axis=1, keepdims=True)
                oh = iota == sel
                ohs.append(oh)
                ds.append(mk)
                if k < 2:
                    cur = jnp.where(oh, f32(2e10), cur)
            empty = ds[0] >= f32(1e9)
            r0 = 1.0 / (ds[0] + f32(1e-8))
            r1 = 1.0 / (ds[1] + f32(1e-8))
            r2 = 1.0 / (ds[2] + f32(1e-8))
            nrm = jnp.maximum(r0 + r1 + r2, f32(1e-8))
            w0 = r0 / nrm
            w1 = r1 / nrm
            w2 = r2 / nrm
            g0 = jnp.dot(ohs[0].astype(f32), gmat, preferred_element_type=f32)
            g1 = jnp.dot(ohs[1].astype(f32), gmat, preferred_element_type=f32)
            g2 = jnp.dot(ohs[2].astype(f32), gmat, preferred_element_type=f32)
            interp = (w0 * g0[:, 3:] + w1 * g1[:, 3:]) + w2 * g2[:, 3:]
            c3 = jnp.concatenate([cx, cy, cz], axis=1)
            l0 = c3 - g0[:, 0:3]
            l1 = c3 - g1[:, 0:3]
            l2 = c3 - g2[:, 0:3]
            fv = jnp.concatenate([interp, l0, l1, l2], axis=1)
            fv = jnp.where(empty, f32(0.0), fv)
            Wv = wsep_refs[g][v]
            outs_v.append(jnp.maximum(jnp.dot(fv, Wv, preferred_element_type=f32)
                                      * f32(_INV), f32(0.0)))
        og = jnp.concatenate(outs_v, axis=1)
        for wp in wpost_refs[g]:
            og = jnp.maximum(jnp.dot(og, wp[...], preferred_element_type=f32)
                             * f32(_INV), f32(0.0))
        group_outs.append(og)

    cat = jnp.concatenate(group_outs + [q], axis=1)
    out = jnp.maximum(jnp.dot(cat, Wmsg_ref[...], preferred_element_type=f32)
                      * f32(_INV), f32(0.0))
    out_ref[...] = out


def kernel(xyz, xyz_batch_cnt, new_xyz, new_xyz_batch_cnt, features,
           W_sep0, W_post0_0, W_post0_1, W_sep1, W_post1_0, W_post1_1, W_msg0):
    del xyz_batch_cnt, new_xyz_batch_cnt
    xyzT = xyz.T
    nchunks = _QB // _QC
    grid = (_B, nchunks)
    out = pl.pallas_call(
        _body,
        grid=grid,
        in_specs=[
            pl.BlockSpec((3, _SB), lambda b, c: (0, b)),
            pl.BlockSpec((_SB, 3), lambda b, c: (b, 0)),
            pl.BlockSpec((_SB, _C_IN), lambda b, c: (b, 0)),
            pl.BlockSpec((_QC, 3), lambda b, c: (b * nchunks + c, 0)),
            pl.BlockSpec(W_sep0.shape, lambda b, c: (0, 0, 0)),
            pl.BlockSpec(W_post0_0.shape, lambda b, c: (0, 0)),
            pl.BlockSpec(W_post0_1.shape, lambda b, c: (0, 0)),
            pl.BlockSpec(W_sep1.shape, lambda b, c: (0, 0, 0)),
            pl.BlockSpec(W_post1_0.shape, lambda b, c: (0, 0)),
            pl.BlockSpec(W_post1_1.shape, lambda b, c: (0, 0)),
            pl.BlockSpec(W_msg0.shape, lambda b, c: (0, 0)),
        ],
        out_specs=pl.BlockSpec((_QC, 128), lambda b, c: (b * nchunks + c, 0)),
        out_shape=jax.ShapeDtypeStruct((_M, 128), jnp.float32),
    )(xyzT, xyz, features, new_xyz,
      W_sep0, W_post0_0, W_post0_1, W_sep1, W_post1_0, W_post1_1, W_msg0)
    return new_xyz, out


# speedup vs baseline: 1.1932x; 1.1932x over previous
import numpy as np
import jax
import jax.numpy as jnp
from jax.experimental import pallas as pl
from jax.experimental.pallas import tpu as pltpu

_N = 8192
_M = 2048
_B = 4
_C_IN = 128
_C_RED = 32
_C_LOCAL = 32
_GROUP_CFGS = (
    ((2, 2, 2), 0.2, (32, 32)),
    ((3, 3, 3), 0.4, (32, 32)),
)
_MULT = 2.0
_BN_EPS = 1e-5
_SB = _N // _B
_QB = _M // _B
_QC = 256
_INV = float(1.0 / np.sqrt(np.float32(1.0 + _BN_EPS)))
_BIG = np.float32(1e10)


def _grid_offsets_np(R, nv):
    gs = [np.arange(-R + R / n, R - R / n + 1e-5, 2 * R / n, dtype=np.float32)
          for n in nv]
    gx, gy, gz = np.meshgrid(gs[0], gs[1], gs[2], indexing='ij')
    return np.stack([gx.reshape(-1), gy.reshape(-1), gz.reshape(-1)], axis=-1)


def _body(xyzT_ref, xyz_ref, feat_ref, q_ref,
          Wsep0_ref, Wp00_ref, Wp01_ref, Wsep1_ref, Wp10_ref, Wp11_ref,
          Wmsg_ref, out_ref):
    f32 = jnp.float32
    sx = xyzT_ref[0:1, :]
    sy = xyzT_ref[1:2, :]
    sz = xyzT_ref[2:3, :]
    q = q_ref[...]
    qx = q[:, 0:1]
    qy = q[:, 1:2]
    qz = q[:, 2:3]
    feats = feat_ref[...]
    fred = (feats[:, 0:32] + feats[:, 32:64]
            + feats[:, 64:96] + feats[:, 96:128])
    gmat = jnp.concatenate([xyz_ref[...], fred], axis=1)

    cheb = jnp.maximum(jnp.maximum(jnp.abs(qx - sx), jnp.abs(qy - sy)),
                       jnp.abs(qz - sz))
    iota = jax.lax.broadcasted_iota(jnp.int32, (_QC, _SB), 1)

    group_outs = []
    wsep_refs = (Wsep0_ref, Wsep1_ref)
    wpost_refs = ((Wp00_ref, Wp01_ref), (Wp10_ref, Wp11_ref))
    for g, (nv, R, _post) in enumerate(_GROUP_CFGS):
        tv = nv[0] * nv[1] * nv[2]
        offs = _grid_offsets_np(R, nv)
        cand = cheb <= f32(R * _MULT)
        ovals = [float(o) for o in np.unique(offs[:, 0])]
        dxyz = (qx - sx, qy - sy, qz - sz)
        terms = []
        for d in dxyz:
            d2ax = d * d
            terms.append({o: d2ax + (f32(2.0 * o) * d + f32(o * o))
                          for o in ovals})
        outs_v = []
        for v in range(tv):
            ox = float(offs[v, 0])
            oy = float(offs[v, 1])
            oz = float(offs[v, 2])
            cx = qx + f32(ox)
            cy = qy + f32(oy)
            cz = qz + f32(oz)
            d2 = (terms[0][ox] + terms[1][oy]) + terms[2][oz]
            cur = jnp.where(cand, d2, _BIG)
            ohs = []
            ds = []
            for k in range(3):
                mk = jnp.min(cur, axis=1, keepdims=True)
                sel = jnp.min(jnp.where(cur == mk, iota, _SB),
                              axis=1, keepdims=True)
                oh = iota == sel
                ohs.append(oh)
                ds.append(mk)
                if k < 2:
                    cur = jnp.where(oh, f32(2e10), cur)
            empty = ds[0] >= f32(1e9)
            r0 = 1.0 / (ds[0] + f32(1e-8))
            r1 = 1.0 / (ds[1] + f32(1e-8))
            r2 = 1.0 / (ds[2] + f32(1e-8))
            nrm = jnp.maximum(r0 + r1 + r2, f32(1e-8))
            w0 = r0 / nrm
            w1 = r1 / nrm
            w2 = r2 / nrm
            g0 = jnp.dot(ohs[0].astype(f32), gmat, preferred_element_type=f32)
            g1 = jnp.dot(ohs[1].astype(f32), gmat, preferred_element_type=f32)
            g2 = jnp.dot(ohs[2].astype(f32), gmat, preferred_element_type=f32)
            interp = (w0 * g0[:, 3:] + w1 * g1[:, 3:]) + w2 * g2[:, 3:]
            c3 = jnp.concatenate([cx, cy, cz], axis=1)
            l0 = c3 - g0[:, 0:3]
            l1 = c3 - g1[:, 0:3]
            l2 = c3 - g2[:, 0:3]
            fv = jnp.concatenate([interp, l0, l1, l2], axis=1)
            fv = jnp.where(empty, f32(0.0), fv)
            Wv = wsep_refs[g][v]
            outs_v.append(jnp.maximum(jnp.dot(fv, Wv, preferred_element_type=f32)
                                      * f32(_INV), f32(0.0)))
        og = jnp.concatenate(outs_v, axis=1)
        for wp in wpost_refs[g]:
            og = jnp.maximum(jnp.dot(og, wp[...], preferred_element_type=f32)
                             * f32(_INV), f32(0.0))
        group_outs.append(og)

    cat = jnp.concatenate(group_outs + [q], axis=1)
    out = jnp.maximum(jnp.dot(cat, Wmsg_ref[...], preferred_element_type=f32)
                      * f32(_INV), f32(0.0))
    out_ref[...] = out


def kernel(xyz, xyz_batch_cnt, new_xyz, new_xyz_batch_cnt, features,
           W_sep0, W_post0_0, W_post0_1, W_sep1, W_post1_0, W_post1_1, W_msg0):
    del xyz_batch_cnt, new_xyz_batch_cnt
    xyzT = xyz.T
    nchunks = _QB // _QC
    grid = (_B, nchunks)
    out = pl.pallas_call(
        _body,
        grid=grid,
        in_specs=[
            pl.BlockSpec((3, _SB), lambda b, c: (0, b)),
            pl.BlockSpec((_SB, 3), lambda b, c: (b, 0)),
            pl.BlockSpec((_SB, _C_IN), lambda b, c: (b, 0)),
            pl.BlockSpec((_QC, 3), lambda b, c: (b * nchunks + c, 0)),
            pl.BlockSpec(W_sep0.shape, lambda b, c: (0, 0, 0)),
            pl.BlockSpec(W_post0_0.shape, lambda b, c: (0, 0)),
            pl.BlockSpec(W_post0_1.shape, lambda b, c: (0, 0)),
            pl.BlockSpec(W_sep1.shape, lambda b, c: (0, 0, 0)),
            pl.BlockSpec(W_post1_0.shape, lambda b, c: (0, 0)),
            pl.BlockSpec(W_post1_1.shape, lambda b, c: (0, 0)),
            pl.BlockSpec(W_msg0.shape, lambda b, c: (0, 0)),
        ],
        out_specs=pl.BlockSpec((_QC, 128), lambda b, c: (b * nchunks + c, 0)),
        out_shape=jax.ShapeDtypeStruct((_M, 128), jnp.float32),
        compiler_params=pltpu.CompilerParams(
            dimension_semantics=("parallel", "parallel")),
    )(xyzT, xyz, features, new_xyz,
      W_sep0, W_post0_0, W_post0_1, W_sep1, W_post1_0, W_post1_1, W_msg0)
    return new_xyz, out
